# baseline (speedup 1.0000x reference)
import functools

import jax
import jax.numpy as jnp
from jax import lax
from jax.experimental import pallas as pl
from jax.experimental.pallas import tpu as pltpu
from jax.experimental.pallas import tpu_sc as plsc

BATCH = 16384
FIELDS = 26
EMBEDDING_SIZE = 32

NC = 2
NS = 16
NW = NC * NS

B = BATCH * FIELDS
D = EMBEDDING_SIZE
BPW = B // NW
CHUNK = 512
NCHUNK = BPW // CHUNK
assert BPW * NW == B and NCHUNK * CHUNK == BPW


NBUF = 4


def _gather_kernel(idx_hbm, w_hbm, out_hbm, idx_v, rows_v,
                   gsem0, gsem1, osem0, osem1):
    wid = lax.axis_index("s") * NC + lax.axis_index("c")
    pltpu.sync_copy(idx_hbm.at[wid], idx_v)

    gsems = (gsem0, gsem1)
    osems = (osem0, osem1)

    def gather_chunk(j, slot, par):
        return pltpu.make_async_copy(
            w_hbm.at[idx_v.at[j]], rows_v.at[slot], gsems[par])

    def out_chunk(j, slot, par):
        return pltpu.make_async_copy(
            rows_v.at[slot], out_hbm.at[wid, j], osems[par])

    gather_chunk(0, 0, 0).start()
    gather_chunk(1, 1, 1).start()

    for j in (0, 1):
        gather_chunk(j, j, j % 2).wait()
        gather_chunk(j + 2, j + 2, j % 2).start()
        out_chunk(j, j, j % 2).start()

    def step(j, par):
        slot = lax.rem(j, NBUF)
        gather_chunk(j, slot, par).wait()
        out_chunk(j - 2, lax.rem(j - 2, NBUF), par).wait()
        gather_chunk(j + 2, lax.rem(j + 2, NBUF), par).start()
        out_chunk(j, slot, par).start()

    def body(i, _):
        j = 2 + 2 * i
        step(j, 0)
        step(j + 1, 1)
        return 0

    assert (NCHUNK - 4) % 2 == 0 and NCHUNK >= 6
    lax.fori_loop(0, (NCHUNK - 4) // 2, body, 0)

    for j in (NCHUNK - 2, NCHUNK - 1):
        gather_chunk(j, j % NBUF, j % 2).wait()
        out_chunk(j - 2, (j - 2) % NBUF, j % 2).wait()
        out_chunk(j, j % NBUF, j % 2).start()
    for j in (NCHUNK - 2, NCHUNK - 1):
        out_chunk(j, j % NBUF, j % 2).wait()


@jax.jit
def kernel(inputs, weight):
    idx = inputs.astype(jnp.int32).reshape(NW, NCHUNK, CHUNK)
    mesh = plsc.VectorSubcoreMesh(core_axis_name="c", subcore_axis_name="s")
    out = pl.kernel(
        _gather_kernel,
        out_type=jax.ShapeDtypeStruct((NW, NCHUNK, CHUNK, D), jnp.float32),
        mesh=mesh,
        scratch_types=[
            pltpu.VMEM((NCHUNK, CHUNK), jnp.int32),
            pltpu.VMEM((NBUF, CHUNK, D), jnp.float32),
            pltpu.SemaphoreType.DMA,
            pltpu.SemaphoreType.DMA,
            pltpu.SemaphoreType.DMA,
            pltpu.SemaphoreType.DMA,
        ],
        compiler_params=pltpu.CompilerParams(use_tc_tiling_on_sc=False),
    )(idx, weight)
    return out.reshape(BATCH, FIELDS, D)
